# baseline (speedup 1.0000x reference)
import functools

import jax
import jax.numpy as jnp
from jax.experimental import pallas as pl
from jax.experimental.pallas import tpu as pltpu

_NUM_DBOX = 8732
_PAD = 8960
_NUM_CLASSES = 21
_NUM_OBJS = 16
_JACCARD_THRESH = 0.5
_NEGPOS_RATIO = 3
_VAR0 = 0.1
_VAR1 = 0.2
_LS = 0.05
_MAX_NEG = 64
_INT_MIN = -2147483648
_INT_MAX = 2147483647


def _f32_sortkey(x):
    b = jax.lax.bitcast_convert_type(x, jnp.int32)
    m = jax.lax.shift_right_arithmetic(b, 31)
    return b ^ (m & jnp.int32(0x7FFFFFFF))


def _key_to_f32(t):
    m = jax.lax.shift_right_arithmetic(t, 31)
    return jax.lax.bitcast_convert_type(t ^ (m & jnp.int32(0x7FFFFFFF)), jnp.float32)


def _blend16(bits, cols):
    vals = cols
    for bit in bits:
        vals = [jnp.where(bit, vals[2 * j + 1], vals[2 * j])
                for j in range(len(vals) // 2)]
    return vals[0]


def _mbl_kernel(nimg, tgt_ref, dbox_ref, loc_ref, conf_ref,
                ll_ref, lc_ref, np_ref, nn_ref):
    i = pl.program_id(0)

    @pl.when(i == 0)
    def _init():
        ll_ref[0, 0] = 0.0
        lc_ref[0, 0] = 0.0
        np_ref[0, 0] = 0.0
        nn_ref[0, 0] = 0.0

    B = nimg
    tgt = tgt_ref[...]

    dcx = dbox_ref[0:1, :]
    dcy = dbox_ref[1:2, :]
    dw = dbox_ref[2:3, :]
    dh = dbox_ref[3:4, :]

    bx1 = dcx - dw / 2.0
    by1 = dcy - dh / 2.0
    bx2 = dcx + dw / 2.0
    by2 = dcy + dh / 2.0
    area_b = (bx2 - bx1) * (by2 - by1)

    lane = jax.lax.broadcasted_iota(jnp.int32, (B, _PAD), 1)

    bto = None
    bti = None
    bpi_list = []
    for t in range(_NUM_OBJS):
        ax1 = tgt[:, t, 0:1]
        ay1 = tgt[:, t, 1:2]
        ax2 = tgt[:, t, 2:3]
        ay2 = tgt[:, t, 3:4]
        iw = jnp.maximum(jnp.minimum(ax2, bx2) - jnp.maximum(ax1, bx1), 0.0)
        ih = jnp.maximum(jnp.minimum(ay2, by2) - jnp.maximum(ay1, by1), 0.0)
        inter = iw * ih
        area_a = (ax2 - ax1) * (ay2 - ay1)
        ov = inter / ((area_a + area_b) - inter)
        if t == 0:
            bto = ov
            bti = jnp.zeros((B, _PAD), jnp.int32)
        else:
            m = ov > bto
            bto = jnp.where(m, ov, bto)
            bti = jnp.where(m, t, bti)
        bpo_t = jnp.max(ov, axis=1, keepdims=True)
        bpi_t = jnp.min(jnp.where(ov == bpo_t, lane, _PAD),
                        axis=1, keepdims=True)
        bpi_list.append(bpi_t)

    for t in range(_NUM_OBJS):
        eq = lane == bpi_list[t]
        bto = jnp.where(eq, 2.0, bto)
        bti = jnp.where(eq, t, bti)

    one = jnp.int32(1)
    bits = [(bti & (one << s)) > 0 for s in range(4)]
    mx1 = _blend16(bits, [tgt[:, t, 0:1] for t in range(16)])
    my1 = _blend16(bits, [tgt[:, t, 1:2] for t in range(16)])
    mx2 = _blend16(bits, [tgt[:, t, 2:3] for t in range(16)])
    my2 = _blend16(bits, [tgt[:, t, 3:4] for t in range(16)])
    mlab = _blend16(bits, [tgt[:, t, 4:5] for t in range(16)])

    clab = jnp.where(bto < _JACCARD_THRESH, 0, mlab.astype(jnp.int32))
    valid = lane < _NUM_DBOX
    pos = (clab > 0) & valid
    posf = pos.astype(jnp.float32)

    g_cx = ((mx1 + mx2) / 2.0 - dcx) / (_VAR0 * dw)
    g_cy = ((my1 + my2) / 2.0 - dcy) / (_VAR0 * dh)
    g_w = jnp.log((mx2 - mx1) / dw) / _VAR1
    g_h = jnp.log((my2 - my1) / dh) / _VAR1

    sl1 = jnp.zeros((B, _PAD), jnp.float32)
    for c, g in enumerate((g_cx, g_cy, g_w, g_h)):
        d = loc_ref[:, c, :] - g
        ad = jnp.abs(d)
        sl1 = sl1 + jnp.where(ad < 1.0, 0.5 * d * d, ad - 0.5)
    ll_img = jnp.sum(sl1 * posf)

    rows = [conf_ref[:, c, :] for c in range(_NUM_CLASSES)]
    cmax = rows[0]
    for c in range(1, _NUM_CLASSES):
        cmax = jnp.maximum(cmax, rows[c])
    sumex = jnp.zeros((B, _PAD), jnp.float32)
    sumc = jnp.zeros((B, _PAD), jnp.float32)
    at_lab = jnp.zeros((B, _PAD), jnp.float32)
    for c in range(_NUM_CLASSES):
        sh_c = rows[c] - cmax
        sumex = sumex + jnp.exp(sh_c)
        sumc = sumc + sh_c
        at_lab = at_lab + jnp.where(clab == c, sh_c, 0.0)
    lse = jnp.log(sumex)
    nll = lse - at_lab
    smooth = lse - sumc / _NUM_CLASSES
    loss_c_all = (1.0 - _LS) * nll + _LS * smooth

    lcm = jnp.where(pos, 0.0, loss_c_all)
    lcm_sel = jnp.where(valid, lcm, -1e30)

    np_col = jnp.sum(posf, axis=1, keepdims=True)
    npi = np_col.astype(jnp.int32)
    k = jnp.minimum(npi * _NEGPOS_RATIO, _MAX_NEG)
    k = jnp.where(npi == 0, 32, k)
    k = jnp.minimum(k, _NUM_DBOX)

    skey = _f32_sortkey(lcm_sel)

    lo = jnp.full((B, 1), _INT_MIN, jnp.int32)
    hi = jnp.full((B, 1), _INT_MAX, jnp.int32)
    for _ in range(32):
        mid = jax.lax.shift_right_arithmetic(lo, 1) + \
            jax.lax.shift_right_arithmetic(hi, 1) + (lo & hi & 1)
        cnt = jnp.sum((skey >= mid).astype(jnp.int32), axis=1, keepdims=True)
        sat = cnt >= k
        lo = jnp.where(sat, mid, lo)
        hi = jnp.where(sat, hi, mid)
    t_val = _key_to_f32(lo)
    gt = skey > lo
    cnt_gt = jnp.sum(gt.astype(jnp.int32), axis=1, keepdims=True)
    sum_gt = jnp.sum(jnp.where(gt, lcm, 0.0), axis=1, keepdims=True)
    topk = sum_gt + (k - cnt_gt).astype(jnp.float32) * t_val

    pos_loss = jnp.sum(jnp.where(pos, loss_c_all, 0.0))
    lc_img = pos_loss + jnp.sum(topk)

    ll_ref[0, 0] += ll_img
    lc_ref[0, 0] += lc_img
    np_ref[0, 0] += jnp.sum(np_col)
    nn_ref[0, 0] += jnp.sum(k.astype(jnp.float32))


@functools.partial(jax.jit, static_argnames=("interpret",))
def kernel(loc_data, conf_data, dbox_list, targets, interpret=False):
    num_batch = loc_data.shape[0]
    nimg = 8 if num_batch % 8 == 0 else 1

    dbox_pad = jnp.concatenate(
        [jnp.full((2, _PAD - _NUM_DBOX), -1000.0, jnp.float32),
         jnp.ones((2, _PAD - _NUM_DBOX), jnp.float32)], axis=0)
    dbox_t = jnp.concatenate([dbox_list.T, dbox_pad], axis=1)

    loc_t = jnp.pad(jnp.moveaxis(loc_data, 2, 1),
                    ((0, 0), (0, 0), (0, _PAD - _NUM_DBOX)))
    conf_t = jnp.pad(jnp.moveaxis(conf_data, 2, 1),
                     ((0, 0), (0, 0), (0, _PAD - _NUM_DBOX)))

    out_spec = pl.BlockSpec((1, 1), lambda i: (0, 0),
                            memory_space=pltpu.SMEM)
    ll, lc, npos, nneg = pl.pallas_call(
        functools.partial(_mbl_kernel, nimg),
        grid=(num_batch // nimg,),
        in_specs=[
            pl.BlockSpec((nimg, _NUM_OBJS, 5), lambda i: (i, 0, 0)),
            pl.BlockSpec((4, _PAD), lambda i: (0, 0)),
            pl.BlockSpec((nimg, 4, _PAD), lambda i: (i, 0, 0)),
            pl.BlockSpec((nimg, _NUM_CLASSES, _PAD), lambda i: (i, 0, 0)),
        ],
        out_specs=[out_spec, out_spec, out_spec, out_spec],
        out_shape=[jax.ShapeDtypeStruct((1, 1), jnp.float32)] * 4,
        compiler_params=pltpu.CompilerParams(
            dimension_semantics=("arbitrary",)),
        interpret=interpret,
    )(targets, dbox_t, loc_t, conf_t)

    ll = ll[0, 0]
    lc = lc[0, 0]
    npos = npos[0, 0]
    nneg = nneg[0, 0]
    n_pos = jnp.maximum(npos, 1.0)
    loss_l_out = ll / n_pos
    sel_neg = jnp.maximum(nneg, 1.0)
    loss_c_out = jnp.where(npos > 0, lc / n_pos, lc / sel_neg)
    return loss_l_out, loss_c_out


# speedup vs baseline: 1.4943x; 1.4943x over previous
import functools

import jax
import jax.numpy as jnp
from jax.experimental import pallas as pl
from jax.experimental.pallas import tpu as pltpu

_NUM_DBOX = 8732
_PAD = 8960
_NUM_CLASSES = 21
_NUM_OBJS = 16
_JACCARD_THRESH = 0.5
_NEGPOS_RATIO = 3
_VAR0 = 0.1
_VAR1 = 0.2
_LS = 0.05
_MAX_NEG = 64
_INT_MIN = -2147483648
_INT_MAX = 2147483647


def _f32_sortkey(x):
    b = jax.lax.bitcast_convert_type(x, jnp.int32)
    m = jax.lax.shift_right_arithmetic(b, 31)
    return b ^ (m & jnp.int32(0x7FFFFFFF))


def _key_to_f32(t):
    m = jax.lax.shift_right_arithmetic(t, 31)
    return jax.lax.bitcast_convert_type(t ^ (m & jnp.int32(0x7FFFFFFF)), jnp.float32)


def _blend16(bits, cols):
    vals = cols
    for bit in bits:
        vals = [jnp.where(bit, vals[2 * j + 1], vals[2 * j])
                for j in range(len(vals) // 2)]
    return vals[0]


def _mbl_kernel(nimg, tgt_ref, dbox_ref, loc_ref, conf_ref,
                ll_ref, lc_ref, np_ref, nn_ref):
    i = pl.program_id(0)

    @pl.when(i == 0)
    def _init():
        ll_ref[0, 0] = 0.0
        lc_ref[0, 0] = 0.0
        np_ref[0, 0] = 0.0
        nn_ref[0, 0] = 0.0

    B = nimg
    tgt = tgt_ref[...]
    ax1 = tgt[:, :, 0:1]
    ay1 = tgt[:, :, 1:2]
    ax2 = tgt[:, :, 2:3]
    ay2 = tgt[:, :, 3:4]

    dcx = dbox_ref[0:1, :]
    dcy = dbox_ref[1:2, :]
    dw = dbox_ref[2:3, :]
    dh = dbox_ref[3:4, :]

    bx1 = (dcx - dw / 2.0).reshape(1, 1, _PAD)
    by1 = (dcy - dh / 2.0).reshape(1, 1, _PAD)
    bx2 = (dcx + dw / 2.0).reshape(1, 1, _PAD)
    by2 = (dcy + dh / 2.0).reshape(1, 1, _PAD)

    iw = jnp.maximum(jnp.minimum(ax2, bx2) - jnp.maximum(ax1, bx1), 0.0)
    ih = jnp.maximum(jnp.minimum(ay2, by2) - jnp.maximum(ay1, by1), 0.0)
    inter = iw * ih
    area_a = (ax2 - ax1) * (ay2 - ay1)
    area_b = (bx2 - bx1) * (by2 - by1)
    ov = inter / ((area_a + area_b) - inter)

    ti = jax.lax.broadcasted_iota(jnp.int32, (B, _NUM_OBJS, _PAD), 1)
    li = jax.lax.broadcasted_iota(jnp.int32, (B, _NUM_OBJS, _PAD), 2)

    bto = jnp.max(ov, axis=1)
    bto3 = bto.reshape(B, 1, _PAD)
    bti = jnp.min(jnp.where(ov == bto3, ti, _NUM_OBJS), axis=1)

    bpo = jnp.max(ov, axis=2, keepdims=True)
    bpi = jnp.min(jnp.where(ov == bpo, li, _PAD), axis=2, keepdims=True)

    forced_t = jnp.max(jnp.where(li == bpi, ti, -1), axis=1)
    anyf = forced_t >= 0
    bto = jnp.where(anyf, 2.0, bto)
    bti = jnp.where(anyf, forced_t, bti)

    tcx = (ax1 + ax2) / 2.0
    tcy = (ay1 + ay2) / 2.0
    ltw = jnp.log(ax2 - ax1)
    lth = jnp.log(ay2 - ay1)

    one = jnp.int32(1)
    bits = [(bti & (one << s)) > 0 for s in range(4)]
    col = lambda a, t: a[:, t, :]
    mcx = _blend16(bits, [col(tcx, t) for t in range(16)])
    mcy = _blend16(bits, [col(tcy, t) for t in range(16)])
    mlw = _blend16(bits, [col(ltw, t) for t in range(16)])
    mlh = _blend16(bits, [col(lth, t) for t in range(16)])
    mlab = _blend16(bits, [tgt[:, t, 4:5] for t in range(16)])

    clab = jnp.where(bto < _JACCARD_THRESH, 0, mlab.astype(jnp.int32))
    lane = jax.lax.broadcasted_iota(jnp.int32, (B, _PAD), 1)
    valid = lane < _NUM_DBOX
    pos = (clab > 0) & valid
    posf = pos.astype(jnp.float32)

    dcx2, dcy2, dw2, dh2 = (a.reshape(1, _PAD) for a in (dcx, dcy, dw, dh))
    rw = 1.0 / (_VAR0 * dw2)
    rh = 1.0 / (_VAR0 * dh2)
    ldw = jnp.log(dw2)
    ldh = jnp.log(dh2)
    g_cx = (mcx - dcx2) * rw
    g_cy = (mcy - dcy2) * rh
    g_w = (mlw - ldw) * (1.0 / _VAR1)
    g_h = (mlh - ldh) * (1.0 / _VAR1)

    sl1 = jnp.zeros((B, _PAD), jnp.float32)
    for c, g in enumerate((g_cx, g_cy, g_w, g_h)):
        d = loc_ref[:, c, :] - g
        ad = jnp.abs(d)
        sl1 = sl1 + jnp.where(ad < 1.0, 0.5 * d * d, ad - 0.5)
    ll_img = jnp.sum(sl1 * posf)

    cf = conf_ref[...]
    cmax = jnp.max(cf, axis=1, keepdims=True)
    sh = cf - cmax
    lse = jnp.log(jnp.sum(jnp.exp(sh), axis=1))
    ci = jax.lax.broadcasted_iota(jnp.int32, (B, _NUM_CLASSES, _PAD), 1)
    clab3 = clab.reshape(B, 1, _PAD)
    w_hi = (1.0 - _LS) + _LS / _NUM_CLASSES
    w_lo = _LS / _NUM_CLASSES
    wsum = jnp.sum(jnp.where(ci == clab3, w_hi, w_lo) * sh, axis=1)
    loss_c_all = lse - wsum

    lcm = jnp.where(pos, 0.0, loss_c_all)
    lcm_sel = jnp.where(valid, lcm, -1e30)

    np_col = jnp.sum(posf, axis=1, keepdims=True)
    npi = np_col.astype(jnp.int32)
    k = jnp.minimum(npi * _NEGPOS_RATIO, _MAX_NEG)
    k = jnp.where(npi == 0, 32, k)
    k = jnp.minimum(k, _NUM_DBOX)

    skey = _f32_sortkey(lcm_sel)

    lo = jnp.full((B, 1), _INT_MIN, jnp.int32)
    hi = jnp.full((B, 1), _INT_MAX, jnp.int32)
    for _ in range(32):
        mid = jax.lax.shift_right_arithmetic(lo, 1) + \
            jax.lax.shift_right_arithmetic(hi, 1) + (lo & hi & 1)
        cnt = jnp.sum((skey >= mid).astype(jnp.int32), axis=1, keepdims=True)
        sat = cnt >= k
        lo = jnp.where(sat, mid, lo)
        hi = jnp.where(sat, hi, mid)
    t_val = _key_to_f32(lo)
    gt = skey > lo
    cnt_gt = jnp.sum(gt.astype(jnp.int32), axis=1, keepdims=True)
    sum_gt = jnp.sum(jnp.where(gt, lcm, 0.0), axis=1, keepdims=True)
    topk = sum_gt + (k - cnt_gt).astype(jnp.float32) * t_val

    pos_loss = jnp.sum(jnp.where(pos, loss_c_all, 0.0))
    lc_img = pos_loss + jnp.sum(topk)

    ll_ref[0, 0] += ll_img
    lc_ref[0, 0] += lc_img
    np_ref[0, 0] += jnp.sum(np_col)
    nn_ref[0, 0] += jnp.sum(k.astype(jnp.float32))


@functools.partial(jax.jit, static_argnames=("interpret",))
def kernel(loc_data, conf_data, dbox_list, targets, interpret=False):
    num_batch = loc_data.shape[0]
    nimg = 8 if num_batch % 8 == 0 else 1

    dbox_pad = jnp.concatenate(
        [jnp.full((2, _PAD - _NUM_DBOX), -1000.0, jnp.float32),
         jnp.ones((2, _PAD - _NUM_DBOX), jnp.float32)], axis=0)
    dbox_t = jnp.concatenate([dbox_list.T, dbox_pad], axis=1)

    loc_t = jnp.pad(jnp.moveaxis(loc_data, 2, 1),
                    ((0, 0), (0, 0), (0, _PAD - _NUM_DBOX)))
    conf_t = jnp.pad(jnp.moveaxis(conf_data, 2, 1),
                     ((0, 0), (0, 0), (0, _PAD - _NUM_DBOX)))

    out_spec = pl.BlockSpec((1, 1), lambda i: (0, 0),
                            memory_space=pltpu.SMEM)
    ll, lc, npos, nneg = pl.pallas_call(
        functools.partial(_mbl_kernel, nimg),
        grid=(num_batch // nimg,),
        in_specs=[
            pl.BlockSpec((nimg, _NUM_OBJS, 5), lambda i: (i, 0, 0)),
            pl.BlockSpec((4, _PAD), lambda i: (0, 0)),
            pl.BlockSpec((nimg, 4, _PAD), lambda i: (i, 0, 0)),
            pl.BlockSpec((nimg, _NUM_CLASSES, _PAD), lambda i: (i, 0, 0)),
        ],
        out_specs=[out_spec, out_spec, out_spec, out_spec],
        out_shape=[jax.ShapeDtypeStruct((1, 1), jnp.float32)] * 4,
        compiler_params=pltpu.CompilerParams(
            dimension_semantics=("arbitrary",)),
        interpret=interpret,
    )(targets, dbox_t, loc_t, conf_t)

    ll = ll[0, 0]
    lc = lc[0, 0]
    npos = npos[0, 0]
    nneg = nneg[0, 0]
    n_pos = jnp.maximum(npos, 1.0)
    loss_l_out = ll / n_pos
    sel_neg = jnp.maximum(nneg, 1.0)
    loss_c_out = jnp.where(npos > 0, lc / n_pos, lc / sel_neg)
    return loss_l_out, loss_c_out
